# baseline (speedup 1.0000x reference)
import functools

import jax
import jax.numpy as jnp
from jax import lax
from jax.experimental import pallas as pl
from jax.experimental.pallas import tpu as pltpu
from jax.experimental.pallas import tpu_sc as plsc

N_NODES = 100000
N_EDGES = 3200000
N_PAD = 100352
EDGES_PER_WORKER = 100352
CHUNK = 2048


def _make_sc_kernel(n_pad, epw, chunk, interpret=False):
    nps = n_pad // 16
    psub = nps // 7
    assert psub % 16 == 0 and nps % psub == 0

    def _sc_body(row_hbm, col_hbm, nodes_t_hbm, wt_hbm, out_hbm,
                 lmax, xbuf, wt_v, row_v, col_v, pr_v, pc_v,
                 p_sh, sem, sem2):
        cid = lax.axis_index("c")
        sid = lax.axis_index("s")
        wid = sid * 2 + cid

        pltpu.sync_copy(wt_hbm, wt_v)
        w0 = wt_v[pl.ds(0, 16)]
        w1 = wt_v[pl.ds(16, 16)]
        w2 = wt_v[pl.ds(32, 16)]
        base_n = sid * nps
        for cb in range(nps // psub):
            off = base_n + cb * psub
            pltpu.sync_copy(nodes_t_hbm.at[pl.ds(off, psub)],
                            xbuf.at[pl.ds(0, psub)])
            pltpu.sync_copy(nodes_t_hbm.at[pl.ds(n_pad + off, psub)],
                            xbuf.at[pl.ds(psub, psub)])
            pltpu.sync_copy(nodes_t_hbm.at[pl.ds(2 * n_pad + off, psub)],
                            xbuf.at[pl.ds(2 * psub, psub)])

            def p_body(j, _):
                s = pl.ds(j * 16, 16)
                pr_v[s] = (xbuf[pl.ds(j * 16, 16)] * w0
                           + xbuf[pl.ds(psub + j * 16, 16)] * w1
                           + xbuf[pl.ds(2 * psub + j * 16, 16)] * w2)
                return 0

            lax.fori_loop(0, psub // 16, p_body, 0)
            pltpu.sync_copy(pr_v.at[pl.ds(0, psub)], p_sh.at[pl.ds(off, psub)])

        zeros16 = jnp.zeros((16,), jnp.float32)

        def z_body(i, _):
            lmax[pl.ds(i * 16, 16)] = zeros16
            return 0

        lax.fori_loop(0, n_pad // 16, z_body, 0)

        plsc.subcore_barrier()

        n_chunks = epw // chunk

        def fire_idx(k, pi):
            eoff = wid * epw + lax.rem(k, n_chunks) * chunk
            pltpu.async_copy(row_hbm.at[pl.ds(eoff, chunk)],
                             row_v.at[pl.ds(pi, chunk)], sem2)
            pltpu.async_copy(col_hbm.at[pl.ds(eoff, chunk)],
                             col_v.at[pl.ds(pi, chunk)], sem2)

        def drain_idx():
            pltpu.make_async_copy(row_hbm.at[pl.ds(0, 2 * chunk)],
                                  col_v.at[pl.ds(0, 2 * chunk)], sem2).wait()

        def fire_gathers(pi, pg):
            for i in range(chunk // 128):
                g = pl.ds(pg + i * 128, 128)
                gi = pl.ds(pi + i * 128, 128)
                pltpu.async_copy(p_sh.at[row_v.at[gi]], pr_v.at[g], sem)
                pltpu.async_copy(p_sh.at[col_v.at[gi]], pc_v.at[g], sem)

        def drain_gathers():
            pltpu.make_async_copy(nodes_t_hbm.at[pl.ds(0, 2 * chunk)],
                                  pr_v.at[pl.ds(0, 2 * chunk)], sem).wait()

        def compute_chunk(pi, pg):
            def rmw_body(j, _):
                for u in range(4):
                    jj = j * 4 + u
                    r = row_v[pl.ds(pi + jj * 16, 16)]
                    d = jnp.abs(pr_v[pl.ds(pg + jj * 16, 16)]
                                - pc_v[pl.ds(pg + jj * 16, 16)])
                    cur = plsc.load_gather(lmax, [r])
                    cnt, _ = plsc.scan_count(r)
                    plsc.store_scatter(lmax, [r], jnp.maximum(cur, d),
                                       mask=cnt == 0)
                return 0

            lax.fori_loop(0, chunk // 64, rmw_body, 0)

            def ver_body(j, cmin):
                for u in range(4):
                    jj = j * 4 + u
                    r = row_v[pl.ds(pi + jj * 16, 16)]
                    d = jnp.abs(pr_v[pl.ds(pg + jj * 16, 16)]
                                - pc_v[pl.ds(pg + jj * 16, 16)])
                    chk = plsc.load_gather(lmax, [r])
                    cmin = jnp.minimum(cmin, chk - d)
                return cmin

            cmin = lax.fori_loop(0, chunk // 64, ver_body,
                                 jnp.zeros((16,), jnp.float32))

            @pl.when(jnp.any(cmin < 0.0))
            def _():
                lane = lax.iota(jnp.int32, 16)

                def tk(x, i):
                    return lax.gather(
                        x, i[:, None],
                        lax.GatherDimensionNumbers(
                            offset_dims=(), collapsed_slice_dims=(0,),
                            start_index_map=(0,)),
                        slice_sizes=(1,),
                        mode=lax.GatherScatterMode.PROMISE_IN_BOUNDS)

                def fix_body(j, _):
                    r = row_v[pl.ds(pi + j * 16, 16)]
                    d = jnp.abs(pr_v[pl.ds(pg + j * 16, 16)]
                                - pc_v[pl.ds(pg + j * 16, 16)])
                    rs, ds = plsc.sort_key_val(r, d)
                    for k in (1, 2, 4, 8):
                        idx = jnp.maximum(lane - k, 0)
                        ds = jnp.where(tk(rs, idx) == rs,
                                       jnp.maximum(ds, tk(ds, idx)), ds)
                    nxt = tk(rs, jnp.minimum(lane + 1, 15))
                    last = (rs != nxt) | (lane == 15)
                    cur = plsc.load_gather(lmax, [rs])
                    plsc.store_scatter(lmax, [rs], jnp.maximum(cur, ds),
                                       mask=last)
                    return 0

                lax.fori_loop(0, chunk // 16, fix_body, 0)

        fire_idx(0, 0)
        drain_idx()
        fire_gathers(0, 0)
        fire_idx(1, chunk)

        def chunk_body(k, _):
            pi = lax.rem(k, 3) * chunk
            pi1 = lax.rem(k + 1, 3) * chunk
            pi2 = lax.rem(k + 2, 3) * chunk
            pg = lax.rem(k, 2) * chunk
            pg1 = lax.rem(k + 1, 2) * chunk
            drain_gathers()
            drain_idx()
            fire_gathers(pi1, pg1)
            fire_idx(k + 2, pi2)
            compute_chunk(pi, pg)
            return 0

        lax.fori_loop(0, n_chunks - 1, chunk_body, 0)
        drain_gathers()
        drain_idx()
        compute_chunk(lax.rem(n_chunks - 1, 3) * chunk,
                      lax.rem(n_chunks - 1, 2) * chunk)

        pltpu.sync_copy(lmax, out_hbm.at[pl.ds(wid * n_pad, n_pad)])

    return pl.kernel(
        _sc_body,
        out_type=jax.ShapeDtypeStruct((32 * n_pad,), jnp.float32),
        mesh=plsc.VectorSubcoreMesh(core_axis_name="c", subcore_axis_name="s",
                                    num_cores=2, num_subcores=16),
        scratch_types=[
            pltpu.VMEM((n_pad,), jnp.float32),
            pltpu.VMEM((3 * psub,), jnp.float32),
            pltpu.VMEM((48,), jnp.float32),
            pltpu.VMEM((3 * chunk,), jnp.int32),
            pltpu.VMEM((3 * chunk,), jnp.int32),
            pltpu.VMEM((2 * chunk,), jnp.float32),
            pltpu.VMEM((2 * chunk,), jnp.float32),
            pltpu.VMEM_SHARED((n_pad,), jnp.float32),
            pltpu.SemaphoreType.DMA,
            pltpu.SemaphoreType.DMA,
        ],
        compiler_params=pltpu.CompilerParams(needs_layout_passes=False),
        interpret=interpret,
    )


def _tc_body(n_nodes, prev_ref, part_ref, wphi_ref, out_ref):
    c = jnp.sum(wphi_ref[0, :]) * (0.5 / 128.0)
    m = jnp.max(part_ref[:, :n_nodes], axis=0, keepdims=True)
    out_ref[...] = prev_ref[...] * 0.5 + m * c


def _run(prev, nodes, adj, W_phi, W_theta,
         n_nodes=N_NODES, n_edges=N_EDGES, n_pad=N_PAD,
         epw=EDGES_PER_WORKER, chunk=CHUNK, interpret=False):
    e_pad = 32 * epw
    rows = adj[0]
    cols = adj[1]
    rows = jnp.concatenate([rows, jnp.zeros((e_pad - n_edges,), jnp.int32)])
    cols = jnp.concatenate([cols, jnp.zeros((e_pad - n_edges,), jnp.int32)])
    nodes_t = jnp.pad(nodes.T, ((0, 0), (0, n_pad - n_nodes))).reshape(3 * n_pad)
    wt_b = jnp.broadcast_to(W_theta.reshape(3, 1), (3, 16)).reshape(48)

    sc = _make_sc_kernel(n_pad, epw, chunk, interpret=interpret)
    partials = sc(rows, cols, nodes_t, wt_b).reshape(32, n_pad)

    out2 = pl.pallas_call(
        functools.partial(_tc_body, n_nodes),
        out_shape=jax.ShapeDtypeStruct((1, n_nodes), jnp.float32),
        interpret=bool(interpret),
    )(prev.reshape(1, n_nodes), partials, W_phi.reshape(1, 128))
    return out2.reshape(n_nodes)


def kernel(previous_inclusion_score, nodes, adjacency_matrix, W_phi, W_theta):
    return _run(previous_inclusion_score, nodes, adjacency_matrix,
                W_phi, W_theta)
